# baseline (device time: 48216 ns/iter reference)
import jax
import jax.numpy as jnp
from jax import lax
from jax.experimental import pallas as pl
from jax.experimental.pallas import tpu as pltpu

N_DEV = 4


def kernel(x, dy):
    k, d = x.shape
    _, f = dy.shape
    ch = d // N_DEV

    def body(x_ref, dy_ref, out_ref, acc_ref, comm_ref, send_sems, recv_sems):
        my = lax.axis_index("i")
        left = (my + N_DEV - 1) % N_DEV
        right = (my + 1) % N_DEV

        barrier_sem = pltpu.get_barrier_semaphore()
        for nbr in (left, right):
            pl.semaphore_signal(
                barrier_sem, inc=1,
                device_id=(nbr,), device_id_type=pl.DeviceIdType.MESH,
            )
        pl.semaphore_wait(barrier_sem, 2)

        acc_ref[:, :] = lax.dot_general(
            x_ref[:, :], dy_ref[:, :],
            (((0,), (0,)), ((), ())),
            preferred_element_type=jnp.float32,
        )

        def chunk(c):
            return acc_ref[pl.ds(c * ch, ch), :]

        comm_ref[3, :, :] = chunk((my + N_DEV - 1) % N_DEV)

        for s in range(N_DEV - 1):
            src_slot = 3 if s == 0 else s - 1
            rdma = pltpu.make_async_remote_copy(
                src_ref=comm_ref.at[src_slot],
                dst_ref=comm_ref.at[s],
                send_sem=send_sems.at[s],
                recv_sem=recv_sems.at[s],
                device_id=(right,),
                device_id_type=pl.DeviceIdType.MESH,
            )
            rdma.start()
            rdma.wait()
            recv_chunk = (my + 2 * N_DEV - s - 2) % N_DEV
            if s < N_DEV - 2:
                comm_ref[s, :, :] = comm_ref[s, :, :] + chunk(recv_chunk)
            else:
                out_ref[:, :] = comm_ref[s, :, :] + chunk(recv_chunk)

    return pl.pallas_call(
        body,
        out_shape=jax.ShapeDtypeStruct((ch, f), jnp.float32),
        in_specs=[
            pl.BlockSpec(memory_space=pltpu.VMEM),
            pl.BlockSpec(memory_space=pltpu.VMEM),
        ],
        out_specs=pl.BlockSpec(memory_space=pltpu.VMEM),
        scratch_shapes=[
            pltpu.VMEM((k, f), jnp.float32),
            pltpu.VMEM((4, ch, f), jnp.float32),
            pltpu.SemaphoreType.DMA((N_DEV - 1,)),
            pltpu.SemaphoreType.DMA((N_DEV - 1,)),
        ],
        compiler_params=pltpu.CompilerParams(collective_id=0),
    )(x, dy)


# device time: 27693 ns/iter; 1.7411x vs baseline; 1.7411x over previous
import jax
import jax.numpy as jnp
from jax import lax
from jax.experimental import pallas as pl
from jax.experimental.pallas import tpu as pltpu

N_DEV = 4
HOPS = N_DEV - 1
NSEG = 2


def kernel(x, dy):
    k, d = x.shape
    _, f = dy.shape
    ch = d // N_DEV
    half = ch // 2
    fseg = f // NSEG

    def body(x_ref, dy_ref, out_ref, xt_ref, acc_ref, commA, commB,
             sendA, recvA, sendB, recvB):
        my = lax.axis_index("i")
        left = (my + N_DEV - 1) % N_DEV
        right = (my + 1) % N_DEV

        barrier_sem = pltpu.get_barrier_semaphore()
        for nbr in (left, right):
            pl.semaphore_signal(
                barrier_sem, inc=1,
                device_id=(nbr,), device_id_type=pl.DeviceIdType.MESH,
            )
        pl.semaphore_wait(barrier_sem, 2)

        xt_ref[:, :] = x_ref[:, :].T

        def gemm_rows(row_start, nrows):
            return lax.dot_general(
                xt_ref[pl.ds(row_start, nrows), :], dy_ref[:, :],
                (((1,), (0,)), ((), ())),
                preferred_element_type=jnp.float32,
            )

        cA0 = (my + N_DEV - 1) % N_DEV
        cB0 = (my + 1) % N_DEV

        acc_ref[pl.ds(cA0 * ch, half), :] = gemm_rows(cA0 * ch, half)
        acc_ref[pl.ds(cB0 * ch + half, half), :] = gemm_rows(cB0 * ch + half, half)
        for g in range(NSEG):
            gc = slice(g * fseg, (g + 1) * fseg)
            commA[3, g, :, :] = acc_ref[pl.ds(cA0 * ch, half), gc]
            commB[3, g, :, :] = acc_ref[pl.ds(cB0 * ch + half, half), gc]

        def mk(comm, send_sems, recv_sems, s, g, dst_dev):
            src_slot = 3 if s == 0 else s - 1
            return pltpu.make_async_remote_copy(
                src_ref=comm.at[src_slot, g],
                dst_ref=comm.at[s, g],
                send_sem=send_sems.at[s, g],
                recv_sem=recv_sems.at[s, g],
                device_id=(dst_dev,),
                device_id_type=pl.DeviceIdType.MESH,
            )

        dA = [[mk(commA, sendA, recvA, s, g, right) for g in range(NSEG)]
              for s in range(HOPS)]
        dB = [[mk(commB, sendB, recvB, s, g, left) for g in range(NSEG)]
              for s in range(HOPS)]

        for g in range(NSEG):
            dA[0][g].start()
            dB[0][g].start()

        c_h0 = (my + 2) % N_DEV
        acc_ref[pl.ds(c_h0 * ch, ch), :] = gemm_rows(c_h0 * ch, ch)
        cA1 = (my + 1) % N_DEV
        acc_ref[pl.ds(cA1 * ch, half), :] = gemm_rows(cA1 * ch, half)
        cB1 = (my + N_DEV - 1) % N_DEV
        acc_ref[pl.ds(cB1 * ch + half, half), :] = gemm_rows(cB1 * ch + half, half)
        acc_ref[pl.ds(my * ch, ch), :] = gemm_rows(my * ch, ch)

        for s in range(HOPS):
            cA = (my + 2 * N_DEV - s - 2) % N_DEV
            cB = (my + s + 2) % N_DEV
            for g in range(NSEG):
                gc = slice(g * fseg, (g + 1) * fseg)
                dA[s][g].wait_recv()
                addA = acc_ref[pl.ds(cA * ch, half), gc]
                if s < HOPS - 1:
                    commA[s, g, :, :] = commA[s, g, :, :] + addA
                    dA[s + 1][g].start()
                else:
                    out_ref[0:half, gc] = commA[s, g, :, :] + addA
                dB[s][g].wait_recv()
                addB = acc_ref[pl.ds(cB * ch + half, half), gc]
                if s < HOPS - 1:
                    commB[s, g, :, :] = commB[s, g, :, :] + addB
                    dB[s + 1][g].start()
                else:
                    out_ref[half:ch, gc] = commB[s, g, :, :] + addB

        for s in range(HOPS):
            for g in range(NSEG):
                dA[s][g].wait_send()
                dB[s][g].wait_send()

    return pl.pallas_call(
        body,
        out_shape=jax.ShapeDtypeStruct((ch, f), jnp.float32),
        in_specs=[
            pl.BlockSpec(memory_space=pltpu.VMEM),
            pl.BlockSpec(memory_space=pltpu.VMEM),
        ],
        out_specs=pl.BlockSpec(memory_space=pltpu.VMEM),
        scratch_shapes=[
            pltpu.VMEM((d, k), jnp.float32),
            pltpu.VMEM((d, f), jnp.float32),
            pltpu.VMEM((4, NSEG, half, fseg), jnp.float32),
            pltpu.VMEM((4, NSEG, half, fseg), jnp.float32),
            pltpu.SemaphoreType.DMA((HOPS, NSEG)),
            pltpu.SemaphoreType.DMA((HOPS, NSEG)),
            pltpu.SemaphoreType.DMA((HOPS, NSEG)),
            pltpu.SemaphoreType.DMA((HOPS, NSEG)),
        ],
        compiler_params=pltpu.CompilerParams(collective_id=0),
    )(x, dy)
